# baseline (speedup 1.0000x reference)
import functools

import jax
import jax.numpy as jnp
from jax import lax
from jax.experimental import pallas as pl
from jax.experimental.pallas import tpu as pltpu
from jax.experimental.pallas import tpu_sc as plsc

HIDDEN = 64
BATCH = 16384
ENT = 1000000
LMBDA = 0.0001
NC = 2
NS = 16
NW = NC * NS
LANES = 16
B2 = 2 * BATCH
PER_W = B2 // NW
CHUNK = 128
NCHUNK = PER_W // CHUNK
KH = HIDDEN // LANES

SLAB = 128
NSLAB = ENT // SLAB
TAIL = ENT - NSLAB * SLAB
SLAB_BASE = NSLAB // NW
SLAB_REM = NSLAB - SLAB_BASE * NW
PAIRS = (SLAB_BASE + 1 + 1) // 2


def _k0_kernel(entT_hbm, tail_hbm, out_hbm,
               slab0_v, slab1_v, trow0_v, trow1_v, tail_v,
               semi0, semi1, semo0, semo1):
    wid = lax.axis_index("s") * NC + lax.axis_index("c")
    start = wid * SLAB_BASE + jnp.minimum(wid, SLAB_REM)
    count = SLAB_BASE + jnp.where(wid < SLAB_REM, 1, 0)
    end = start + count

    lane = lax.broadcasted_iota(jnp.int32, (LANES,), 0)

    def fire_in(s, slab_v, sem):
        col = pl.multiple_of(s * SLAB, SLAB)
        pltpu.async_copy(
            entT_hbm.at[:, pl.ds(col, SLAB)], slab_v, sem)

    def wait_in(slab_v, sem):
        pltpu.make_async_copy(
            entT_hbm.at[:, pl.ds(0, SLAB)], slab_v, sem).wait()

    JU = 8

    def transpose(slab_v, trow_v):
        def jbody(jo, _):
            j0 = jo * JU
            for ju in range(JU):
                idx_j = jnp.full((LANES,), ju, jnp.int32) + j0
                for k in range(KH):
                    idx_d = lane + (k * LANES)
                    g = plsc.load_gather(slab_v.at[...], [idx_d, idx_j])
                    trow_v[j0 + ju, pl.ds(k * LANES, LANES)] = g
            return 0
        lax.fori_loop(0, SLAB // JU, jbody, 0)

    def fire_out(s, trow_v, sem):
        row = pl.multiple_of(s * SLAB, SLAB)
        pltpu.async_copy(trow_v, out_hbm.at[pl.ds(row, SLAB)], sem)

    def wait_out(trow_v, sem):
        pltpu.make_async_copy(
            out_hbm.at[pl.ds(0, SLAB)], trow_v, sem).wait()

    bufs = ((slab0_v, trow0_v, semi0, semo0),
            (slab1_v, trow1_v, semi1, semo1))

    for b in range(2):
        @pl.when(start + b < end)
        def _():
            fire_in(start + b, bufs[b][0], bufs[b][2])

    def pair_body(p, _):
        for b in range(2):
            s = start + p * 2 + b
            slab_v, trow_v, semi, semo = bufs[b]

            @pl.when(s < end)
            def _():
                wait_in(slab_v, semi)

                @pl.when(p > 0)
                def _():
                    wait_out(trow_v, semo)

                transpose(slab_v, trow_v)
                fire_out(s, trow_v, semo)

                @pl.when(s + 2 < end)
                def _():
                    fire_in(s + 2, slab_v, semi)
        return 0

    lax.fori_loop(0, PAIRS, pair_body, 0)
    for b in range(2):
        wait_out(bufs[b][1], bufs[b][3])

    @pl.when(wid == NW - 1)
    def _():
        pltpu.sync_copy(tail_hbm, tail_v)
        pltpu.sync_copy(tail_v, out_hbm.at[pl.ds(NSLAB * SLAB, TAIL)])


_k0_call = functools.partial(
    pl.kernel,
    out_type=jax.ShapeDtypeStruct((ENT, HIDDEN), jnp.float32),
    mesh=plsc.VectorSubcoreMesh(
        core_axis_name="c", subcore_axis_name="s",
        num_cores=NC, num_subcores=NS),
    scratch_types=[
        pltpu.VMEM((HIDDEN, SLAB), jnp.float32),
        pltpu.VMEM((HIDDEN, SLAB), jnp.float32),
        pltpu.VMEM((SLAB, HIDDEN), jnp.float32),
        pltpu.VMEM((SLAB, HIDDEN), jnp.float32),
        pltpu.VMEM((TAIL, HIDDEN), jnp.float32),
        pltpu.SemaphoreType.DMA,
        pltpu.SemaphoreType.DMA,
        pltpu.SemaphoreType.DMA,
        pltpu.SemaphoreType.DMA,
    ],
    compiler_params=pltpu.CompilerParams(use_tc_tiling_on_sc=True,
                                         needs_layout_passes=False),
)(_k0_kernel)


def _k1_kernel(h_hbm, t_hbm, r_hbm, ent_hbm, rel_hbm,
               q_hbm, sq_hbm,
               hidx_v, tidx_v, ridx_v,
               hrow_v, trow_v, rrow_v,
               qbuf_v, sq_v,
               sem0, sem1, sem2):
    wid = lax.axis_index("s") * NC + lax.axis_index("c")
    base = wid * PER_W

    pltpu.sync_copy(h_hbm.at[pl.ds(base, PER_W)], hidx_v)
    pltpu.sync_copy(t_hbm.at[pl.ds(base, PER_W)], tidx_v)
    pltpu.sync_copy(r_hbm.at[pl.ds(base, PER_W)], ridx_v)

    sq_acc = jnp.zeros((LANES,), jnp.float32)

    for c in range(NCHUNK):
        def fire_body(jj, _):
            hv = hidx_v[pl.ds(c * CHUNK + jj * LANES, LANES)]
            tv = tidx_v[pl.ds(c * CHUNK + jj * LANES, LANES)]
            rv = ridx_v[pl.ds(c * CHUNK + jj * LANES, LANES)]
            for u in range(LANES):
                j = jj * LANES + u
                pltpu.async_copy(ent_hbm.at[hv[u]], hrow_v.at[j], sem0)
                pltpu.async_copy(ent_hbm.at[tv[u]], trow_v.at[j], sem1)
                pltpu.async_copy(rel_hbm.at[rv[u]], rrow_v.at[j], sem2)
            return 0

        lax.fori_loop(0, CHUNK // LANES, fire_body, 0)
        dummy = ent_hbm.at[pl.ds(0, CHUNK)]
        pltpu.make_async_copy(dummy, hrow_v, sem0).wait()
        pltpu.make_async_copy(dummy, trow_v, sem1).wait()
        pltpu.make_async_copy(dummy, rrow_v, sem2).wait()

        def item_body(i, sq):
            q = jnp.zeros((LANES,), jnp.float32)
            for k in range(KH):
                h = hrow_v[i, pl.ds(k * LANES, LANES)]
                t = trow_v[i, pl.ds(k * LANES, LANES)]
                r = rrow_v[i, pl.ds(k * LANES, LANES)]
                q = q + h * r * t
                sq = sq + h * h + t * t + r * r
            qbuf_v[i, :] = q
            return sq

        sq_acc = lax.fori_loop(0, CHUNK, item_body, sq_acc)

        pltpu.sync_copy(qbuf_v, q_hbm.at[pl.ds(base + c * CHUNK, CHUNK)])

    sq_v[...] = sq_acc
    pltpu.sync_copy(sq_v, sq_hbm.at[wid])


_k1_call = functools.partial(
    pl.kernel,
    out_type=(
        jax.ShapeDtypeStruct((B2, LANES), jnp.float32),
        jax.ShapeDtypeStruct((NW, LANES), jnp.float32),
    ),
    mesh=plsc.VectorSubcoreMesh(
        core_axis_name="c", subcore_axis_name="s",
        num_cores=NC, num_subcores=NS),
    scratch_types=[
        pltpu.VMEM((PER_W,), jnp.int32),
        pltpu.VMEM((PER_W,), jnp.int32),
        pltpu.VMEM((PER_W,), jnp.int32),
        pltpu.VMEM((CHUNK, HIDDEN), jnp.float32),
        pltpu.VMEM((CHUNK, HIDDEN), jnp.float32),
        pltpu.VMEM((CHUNK, HIDDEN), jnp.float32),
        pltpu.VMEM((CHUNK, LANES), jnp.float32),
        pltpu.VMEM((LANES,), jnp.float32),
        pltpu.SemaphoreType.DMA,
        pltpu.SemaphoreType.DMA,
        pltpu.SemaphoreType.DMA,
    ],
    compiler_params=pltpu.CompilerParams(use_tc_tiling_on_sc=True),
)(_k1_kernel)


def _tc_kernel(q_ref, y_ref, sq_ref, o_ref):
    row = lax.broadcasted_iota(jnp.int32, (128, 8), 0)
    col = lax.broadcasted_iota(jnp.int32, (128, 8), 1)
    fold = (row // LANES == col).astype(jnp.float32)
    scores = jnp.dot(q_ref[...], fold,
                     preferred_element_type=jnp.float32)
    x = -y_ref[...] * scores
    sp = jnp.maximum(x, 0.0) + jnp.log1p(jnp.exp(-jnp.abs(x)))
    loss = jnp.sum(sp) / BATCH
    regul = jnp.sum(sq_ref[...]) / (BATCH * HIDDEN)
    o_ref[0, 0] = loss + LMBDA * regul


_tc_call = pl.pallas_call(
    _tc_kernel,
    out_shape=jax.ShapeDtypeStruct((1, 1), jnp.float32),
    out_specs=pl.BlockSpec(memory_space=pltpu.SMEM),
)


def kernel(pos_h, pos_t, pos_r, neg_h, neg_t, neg_r, pos_y, neg_y,
           ent_emb, rel_emb):
    h_all = jnp.concatenate([pos_h, neg_h]).astype(jnp.int32)
    t_all = jnp.concatenate([pos_t, neg_t]).astype(jnp.int32)
    r_all = jnp.concatenate([pos_r, neg_r]).astype(jnp.int32)
    y_all = jnp.concatenate([pos_y, neg_y]).reshape(B2 // 8, 8)

    ent_rm = _k0_call(ent_emb.T, ent_emb[NSLAB * SLAB:])
    q, sq = _k1_call(h_all, t_all, r_all, ent_rm, rel_emb)
    loss = _tc_call(q.reshape(B2 // 8, 128), y_all,
                    sq.reshape(NW * LANES // 128, 128))
    return loss[0, 0]


# speedup vs baseline: 1.1840x; 1.1840x over previous
import functools

import jax
import jax.numpy as jnp
from jax import lax
from jax.experimental import pallas as pl
from jax.experimental.pallas import tpu as pltpu
from jax.experimental.pallas import tpu_sc as plsc

HIDDEN = 64
BATCH = 16384
ENT = 1000000
LMBDA = 0.0001
NC = 2
NS = 16
NW = NC * NS
LANES = 16
B2 = 2 * BATCH
PER_W = B2 // NW
CHUNK = 128
NCHUNK = PER_W // CHUNK
KH = HIDDEN // LANES

SLAB = 128
NSLAB = ENT // SLAB
TAIL = ENT - NSLAB * SLAB
SLAB_BASE = NSLAB // NW
SLAB_REM = NSLAB - SLAB_BASE * NW
NBUF = 4
STEPS = (SLAB_BASE + 1 + NBUF - 1) // NBUF
SLABW = SLAB * HIDDEN


def _k0_kernel(entT_hbm, tail_hbm, out_hbm,
               slab0_v, slab1_v, slab2_v, slab3_v,
               trow0_v, trow1_v, trow2_v, trow3_v, tail_v,
               si0, si1, si2, si3, so0, so1, so2, so3):
    wid = lax.axis_index("s") * NC + lax.axis_index("c")
    start = wid * SLAB_BASE + jnp.minimum(wid, SLAB_REM)
    count = SLAB_BASE + jnp.where(wid < SLAB_REM, 1, 0)
    end = start + count

    lane = lax.broadcasted_iota(jnp.int32, (LANES,), 0)

    def fire_in(s, slab_v, sem):
        col = pl.multiple_of(s * SLAB, SLAB)
        pltpu.async_copy(entT_hbm.at[:, pl.ds(col, SLAB)], slab_v, sem)

    def wait_in(slab_v, sem):
        pltpu.make_async_copy(
            entT_hbm.at[:, pl.ds(0, SLAB)], slab_v, sem).wait()

    def transpose(slab_v, trow_v):
        def jbody(jo, _):
            for ju in range(4):
                idx_j = jnp.full((LANES,), ju, jnp.int32) + jo * 4
                gs = [plsc.load_gather(slab_v.at[...],
                                       [lane + k * LANES, idx_j])
                      for k in range(KH)]
                for k in range(KH):
                    off = (jo * 4 + ju) * HIDDEN + k * LANES
                    trow_v[pl.ds(off, LANES)] = gs[k]
            return 0
        lax.fori_loop(0, SLAB // 4, jbody, 0)

    def fire_out(s, trow_v, sem):
        off = pl.multiple_of(s * SLABW, 8)
        pltpu.async_copy(trow_v, out_hbm.at[pl.ds(off, SLABW)], sem)

    def wait_out(trow_v, sem):
        pltpu.make_async_copy(
            out_hbm.at[pl.ds(0, SLABW)], trow_v, sem).wait()

    bufs = ((slab0_v, trow0_v, si0, so0),
            (slab1_v, trow1_v, si1, so1),
            (slab2_v, trow2_v, si2, so2),
            (slab3_v, trow3_v, si3, so3))

    for b in range(NBUF):
        @pl.when(start + b < end)
        def _():
            fire_in(start + b, bufs[b][0], bufs[b][2])

    def step_body(p, _):
        for b in range(NBUF):
            s = start + p * NBUF + b
            slab_v, trow_v, semi, semo = bufs[b]

            @pl.when(s < end)
            def _():
                wait_in(slab_v, semi)

                @pl.when(p > 0)
                def _():
                    wait_out(trow_v, semo)

                transpose(slab_v, trow_v)
                fire_out(s, trow_v, semo)

                @pl.when(s + NBUF < end)
                def _():
                    fire_in(s + NBUF, slab_v, semi)
        return 0

    lax.fori_loop(0, STEPS, step_body, 0)
    for b in range(NBUF):
        wait_out(bufs[b][1], bufs[b][3])

    @pl.when(wid == NW - 1)
    def _():
        pltpu.sync_copy(tail_hbm, tail_v)
        for r in range(TAIL):
            for k in range(KH):
                trow0_v[pl.ds(r * HIDDEN + k * LANES, LANES)] = (
                    tail_v[r, pl.ds(k * LANES, LANES)])
        pltpu.sync_copy(trow0_v.at[pl.ds(0, TAIL * HIDDEN)],
                        out_hbm.at[pl.ds(NSLAB * SLABW, TAIL * HIDDEN)])


_k0_call = functools.partial(
    pl.kernel,
    out_type=jax.ShapeDtypeStruct((ENT * HIDDEN,), jnp.float32),
    mesh=plsc.VectorSubcoreMesh(
        core_axis_name="c", subcore_axis_name="s",
        num_cores=NC, num_subcores=NS),
    scratch_types=[
        pltpu.VMEM((HIDDEN, SLAB), jnp.float32),
        pltpu.VMEM((HIDDEN, SLAB), jnp.float32),
        pltpu.VMEM((HIDDEN, SLAB), jnp.float32),
        pltpu.VMEM((HIDDEN, SLAB), jnp.float32),
        pltpu.VMEM((SLABW,), jnp.float32),
        pltpu.VMEM((SLABW,), jnp.float32),
        pltpu.VMEM((SLABW,), jnp.float32),
        pltpu.VMEM((SLABW,), jnp.float32),
        pltpu.VMEM((TAIL, HIDDEN), jnp.float32),
        pltpu.SemaphoreType.DMA,
        pltpu.SemaphoreType.DMA,
        pltpu.SemaphoreType.DMA,
        pltpu.SemaphoreType.DMA,
        pltpu.SemaphoreType.DMA,
        pltpu.SemaphoreType.DMA,
        pltpu.SemaphoreType.DMA,
        pltpu.SemaphoreType.DMA,
    ],
    compiler_params=pltpu.CompilerParams(use_tc_tiling_on_sc=True,
                                         needs_layout_passes=False),
)(_k0_kernel)


def _k1_kernel(h_hbm, t_hbm, r_hbm, ent_hbm, rel_hbm,
               q_hbm, sq_hbm,
               hidx_v, tidx_v, ridx_v,
               hrow_v, trow_v, rrow_v,
               qbuf_v, sq_v,
               sem0, sem1, sem2):
    wid = lax.axis_index("s") * NC + lax.axis_index("c")
    base = wid * PER_W

    pltpu.sync_copy(h_hbm.at[pl.ds(base, PER_W)], hidx_v)
    pltpu.sync_copy(t_hbm.at[pl.ds(base, PER_W)], tidx_v)
    pltpu.sync_copy(r_hbm.at[pl.ds(base, PER_W)], ridx_v)

    sq_acc = jnp.zeros((LANES,), jnp.float32)

    for c in range(NCHUNK):
        def fire_body(jj, _):
            hv = hidx_v[pl.ds(c * CHUNK + jj * LANES, LANES)]
            tv = tidx_v[pl.ds(c * CHUNK + jj * LANES, LANES)]
            rv = ridx_v[pl.ds(c * CHUNK + jj * LANES, LANES)]
            for u in range(LANES):
                j = jj * LANES + u
                pltpu.async_copy(
                    ent_hbm.at[pl.ds(hv[u] * HIDDEN, HIDDEN)],
                    hrow_v.at[pl.ds(j * HIDDEN, HIDDEN)], sem0)
                pltpu.async_copy(
                    ent_hbm.at[pl.ds(tv[u] * HIDDEN, HIDDEN)],
                    trow_v.at[pl.ds(j * HIDDEN, HIDDEN)], sem1)
                pltpu.async_copy(rel_hbm.at[rv[u]],
                                 rrow_v.at[j], sem2)
            return 0

        lax.fori_loop(0, CHUNK // LANES, fire_body, 0)
        dummy = ent_hbm.at[pl.ds(0, CHUNK * HIDDEN)]
        pltpu.make_async_copy(dummy, hrow_v, sem0).wait()
        pltpu.make_async_copy(dummy, trow_v, sem1).wait()
        pltpu.make_async_copy(rel_hbm.at[pl.ds(0, CHUNK)],
                              rrow_v, sem2).wait()

        def item_body(i, sq):
            q = jnp.zeros((LANES,), jnp.float32)
            for k in range(KH):
                h = hrow_v[pl.ds(i * HIDDEN + k * LANES, LANES)]
                t = trow_v[pl.ds(i * HIDDEN + k * LANES, LANES)]
                r = rrow_v[i, pl.ds(k * LANES, LANES)]
                q = q + h * r * t
                sq = sq + h * h + t * t + r * r
            qbuf_v[i, :] = q
            return sq

        sq_acc = lax.fori_loop(0, CHUNK, item_body, sq_acc)

        pltpu.sync_copy(qbuf_v, q_hbm.at[pl.ds(base + c * CHUNK, CHUNK)])

    sq_v[...] = sq_acc
    pltpu.sync_copy(sq_v, sq_hbm.at[wid])


_k1_call = functools.partial(
    pl.kernel,
    out_type=(
        jax.ShapeDtypeStruct((B2, LANES), jnp.float32),
        jax.ShapeDtypeStruct((NW, LANES), jnp.float32),
    ),
    mesh=plsc.VectorSubcoreMesh(
        core_axis_name="c", subcore_axis_name="s",
        num_cores=NC, num_subcores=NS),
    scratch_types=[
        pltpu.VMEM((PER_W,), jnp.int32),
        pltpu.VMEM((PER_W,), jnp.int32),
        pltpu.VMEM((PER_W,), jnp.int32),
        pltpu.VMEM((CHUNK * HIDDEN,), jnp.float32),
        pltpu.VMEM((CHUNK * HIDDEN,), jnp.float32),
        pltpu.VMEM((CHUNK, HIDDEN), jnp.float32),
        pltpu.VMEM((CHUNK, LANES), jnp.float32),
        pltpu.VMEM((LANES,), jnp.float32),
        pltpu.SemaphoreType.DMA,
        pltpu.SemaphoreType.DMA,
        pltpu.SemaphoreType.DMA,
    ],
    compiler_params=pltpu.CompilerParams(use_tc_tiling_on_sc=True),
)(_k1_kernel)


def _tc_kernel(q_ref, y_ref, sq_ref, o_ref):
    row = lax.broadcasted_iota(jnp.int32, (128, 8), 0)
    col = lax.broadcasted_iota(jnp.int32, (128, 8), 1)
    fold = (row // LANES == col).astype(jnp.float32)
    scores = jnp.dot(q_ref[...], fold,
                     preferred_element_type=jnp.float32)
    x = -y_ref[...] * scores
    sp = jnp.maximum(x, 0.0) + jnp.log1p(jnp.exp(-jnp.abs(x)))
    loss = jnp.sum(sp) / BATCH
    regul = jnp.sum(sq_ref[...]) / (BATCH * HIDDEN)
    o_ref[0, 0] = loss + LMBDA * regul


_tc_call = pl.pallas_call(
    _tc_kernel,
    out_shape=jax.ShapeDtypeStruct((1, 1), jnp.float32),
    out_specs=pl.BlockSpec(memory_space=pltpu.SMEM),
)


def kernel(pos_h, pos_t, pos_r, neg_h, neg_t, neg_r, pos_y, neg_y,
           ent_emb, rel_emb):
    h_all = jnp.concatenate([pos_h, neg_h]).astype(jnp.int32)
    t_all = jnp.concatenate([pos_t, neg_t]).astype(jnp.int32)
    r_all = jnp.concatenate([pos_r, neg_r]).astype(jnp.int32)
    y_all = jnp.concatenate([pos_y, neg_y]).reshape(B2 // 8, 8)

    ent_rm = _k0_call(ent_emb.T, ent_emb[NSLAB * SLAB:])
    q, sq = _k1_call(h_all, t_all, r_all, ent_rm, rel_emb)
    loss = _tc_call(q.reshape(B2 // 8, 128), y_all,
                    sq.reshape(NW * LANES // 128, 128))
    return loss[0, 0]


# speedup vs baseline: 2.3229x; 1.9619x over previous
import functools

import jax
import jax.numpy as jnp
from jax import lax
from jax.experimental import pallas as pl
from jax.experimental.pallas import tpu as pltpu
from jax.experimental.pallas import tpu_sc as plsc

HIDDEN = 64
BATCH = 16384
ENT = 1000000
LMBDA = 0.0001
NC = 2
NS = 16
NW = NC * NS
LANES = 16
B2 = 2 * BATCH
PER_W = B2 // NW
CHUNK = 128
NCHUNK = PER_W // CHUNK
KH = HIDDEN // LANES
ROWPAIR = 2 * HIDDEN


def _sc_kernel(h_hbm, t_hbm, r_hbm, ent2_hbm, rel_hbm,
               q_hbm, sq_hbm,
               hidx_v, tidx_v, ridx_v,
               hrow_v, trow_v, rrow_v,
               qbuf_v, sq_v,
               sem0, sem1, sem2):
    wid = lax.axis_index("s") * NC + lax.axis_index("c")
    base = wid * PER_W

    pltpu.sync_copy(h_hbm.at[pl.ds(base, PER_W)], hidx_v)
    pltpu.sync_copy(t_hbm.at[pl.ds(base, PER_W)], tidx_v)
    pltpu.sync_copy(r_hbm.at[pl.ds(base, PER_W)], ridx_v)

    sq_acc = jnp.zeros((LANES,), jnp.float32)

    for c in range(NCHUNK):
        def fire_body(jj, _):
            hv = hidx_v[pl.ds(c * CHUNK + jj * LANES, LANES)]
            tv = tidx_v[pl.ds(c * CHUNK + jj * LANES, LANES)]
            rv = ridx_v[pl.ds(c * CHUNK + jj * LANES, LANES)]
            for u in range(LANES):
                j = jj * LANES + u
                pltpu.async_copy(ent2_hbm.at[hv[u] // 2],
                                 hrow_v.at[j], sem0)
                pltpu.async_copy(ent2_hbm.at[tv[u] // 2],
                                 trow_v.at[j], sem1)
                pltpu.async_copy(rel_hbm.at[rv[u]], rrow_v.at[j], sem2)
            return 0

        lax.fori_loop(0, CHUNK // LANES, fire_body, 0)
        pltpu.make_async_copy(
            ent2_hbm.at[pl.ds(0, CHUNK)], hrow_v, sem0).wait()
        pltpu.make_async_copy(
            ent2_hbm.at[pl.ds(0, CHUNK)], trow_v, sem1).wait()
        pltpu.make_async_copy(
            rel_hbm.at[pl.ds(0, CHUNK)], rrow_v, sem2).wait()

        def group_body(jj, sq):
            hv = hidx_v[pl.ds(c * CHUNK + jj * LANES, LANES)]
            tv = tidx_v[pl.ds(c * CHUNK + jj * LANES, LANES)]
            for u in range(LANES):
                i = jj * LANES + u
                hoff = (hv[u] % 2) * HIDDEN
                toff = (tv[u] % 2) * HIDDEN
                q = jnp.zeros((LANES,), jnp.float32)
                for k in range(KH):
                    h = hrow_v[i, pl.ds(hoff + k * LANES, LANES)]
                    t = trow_v[i, pl.ds(toff + k * LANES, LANES)]
                    r = rrow_v[i, pl.ds(k * LANES, LANES)]
                    q = q + h * r * t
                    sq = sq + h * h + t * t + r * r
                qbuf_v[i, :] = q
            return sq

        sq_acc = lax.fori_loop(0, CHUNK // LANES, group_body, sq_acc)

        pltpu.sync_copy(qbuf_v, q_hbm.at[pl.ds(base + c * CHUNK, CHUNK)])

    sq_v[...] = sq_acc
    pltpu.sync_copy(sq_v, sq_hbm.at[wid])


_sc_call = functools.partial(
    pl.kernel,
    out_type=(
        jax.ShapeDtypeStruct((B2, LANES), jnp.float32),
        jax.ShapeDtypeStruct((NW, LANES), jnp.float32),
    ),
    mesh=plsc.VectorSubcoreMesh(
        core_axis_name="c", subcore_axis_name="s",
        num_cores=NC, num_subcores=NS),
    scratch_types=[
        pltpu.VMEM((PER_W,), jnp.int32),
        pltpu.VMEM((PER_W,), jnp.int32),
        pltpu.VMEM((PER_W,), jnp.int32),
        pltpu.VMEM((CHUNK, ROWPAIR), jnp.float32),
        pltpu.VMEM((CHUNK, ROWPAIR), jnp.float32),
        pltpu.VMEM((CHUNK, HIDDEN), jnp.float32),
        pltpu.VMEM((CHUNK, LANES), jnp.float32),
        pltpu.VMEM((LANES,), jnp.float32),
        pltpu.SemaphoreType.DMA,
        pltpu.SemaphoreType.DMA,
        pltpu.SemaphoreType.DMA,
    ],
    compiler_params=pltpu.CompilerParams(use_tc_tiling_on_sc=True),
)(_sc_kernel)


def _tc_kernel(q_ref, y_ref, sq_ref, o_ref):
    row = lax.broadcasted_iota(jnp.int32, (128, 8), 0)
    col = lax.broadcasted_iota(jnp.int32, (128, 8), 1)
    fold = (row // LANES == col).astype(jnp.float32)
    scores = jnp.dot(q_ref[...], fold,
                     preferred_element_type=jnp.float32)
    x = -y_ref[...] * scores
    sp = jnp.maximum(x, 0.0) + jnp.log1p(jnp.exp(-jnp.abs(x)))
    loss = jnp.sum(sp) / BATCH
    regul = jnp.sum(sq_ref[...]) / (BATCH * HIDDEN)
    o_ref[0, 0] = loss + LMBDA * regul


_tc_call = pl.pallas_call(
    _tc_kernel,
    out_shape=jax.ShapeDtypeStruct((1, 1), jnp.float32),
    out_specs=pl.BlockSpec(memory_space=pltpu.SMEM),
)


def kernel(pos_h, pos_t, pos_r, neg_h, neg_t, neg_r, pos_y, neg_y,
           ent_emb, rel_emb):
    h_all = jnp.concatenate([pos_h, neg_h]).astype(jnp.int32)
    t_all = jnp.concatenate([pos_t, neg_t]).astype(jnp.int32)
    r_all = jnp.concatenate([pos_r, neg_r]).astype(jnp.int32)
    y_all = jnp.concatenate([pos_y, neg_y]).reshape(B2 // 8, 8)

    ent2 = ent_emb.reshape(ENT // 2, ROWPAIR)
    q, sq = _sc_call(h_all, t_all, r_all, ent2, rel_emb)
    loss = _tc_call(q.reshape(B2 // 8, 128), y_all,
                    sq.reshape(NW * LANES // 128, 128))
    return loss[0, 0]


# speedup vs baseline: 3.7742x; 1.6248x over previous
import functools

import jax
import jax.numpy as jnp
from jax import lax
from jax.experimental import pallas as pl
from jax.experimental.pallas import tpu as pltpu
from jax.experimental.pallas import tpu_sc as plsc

HIDDEN = 64
BATCH = 16384
LMBDA = 0.0001
NC = 2
NS = 16
NW = NC * NS
LANES = 16
B2 = 2 * BATCH
PER_W = B2 // NW
CHUNK = 128
NCHUNK = PER_W // CHUNK
KH = HIDDEN // LANES


def _sc_kernel(h_hbm, t_hbm, r_hbm, ent_hbm, rel_hbm,
               q_hbm, sq_hbm,
               hidx_v, tidx_v, ridx_v,
               hrow_v, trow_v, rrow_v,
               qbuf_v, sq_v,
               sem0, sem1, sem2):
    wid = lax.axis_index("s") * NC + lax.axis_index("c")
    base = wid * PER_W

    pltpu.sync_copy(h_hbm.at[pl.ds(base, PER_W)], hidx_v)
    pltpu.sync_copy(t_hbm.at[pl.ds(base, PER_W)], tidx_v)
    pltpu.sync_copy(r_hbm.at[pl.ds(base, PER_W)], ridx_v)

    sq_acc = jnp.zeros((LANES,), jnp.float32)

    for c in range(NCHUNK):
        def fire_body(jj, _):
            hv = hidx_v[pl.ds(c * CHUNK + jj * LANES, LANES)]
            tv = tidx_v[pl.ds(c * CHUNK + jj * LANES, LANES)]
            rv = ridx_v[pl.ds(c * CHUNK + jj * LANES, LANES)]
            for u in range(LANES):
                j = jj * LANES + u
                pltpu.async_copy(ent_hbm.at[hv[u]], hrow_v.at[j], sem0)
                pltpu.async_copy(ent_hbm.at[tv[u]], trow_v.at[j], sem1)
                pltpu.async_copy(rel_hbm.at[rv[u]], rrow_v.at[j], sem2)
            return 0

        lax.fori_loop(0, CHUNK // LANES, fire_body, 0)
        dummy = ent_hbm.at[pl.ds(0, CHUNK)]
        pltpu.make_async_copy(dummy, hrow_v, sem0).wait()
        pltpu.make_async_copy(dummy, trow_v, sem1).wait()
        pltpu.make_async_copy(dummy, rrow_v, sem2).wait()

        def item_body(i, sq):
            q = jnp.zeros((LANES,), jnp.float32)
            for k in range(KH):
                h = hrow_v[i, pl.ds(k * LANES, LANES)]
                t = trow_v[i, pl.ds(k * LANES, LANES)]
                r = rrow_v[i, pl.ds(k * LANES, LANES)]
                q = q + h * r * t
                sq = sq + h * h + t * t + r * r
            qbuf_v[i, :] = q
            return sq

        sq_acc = lax.fori_loop(0, CHUNK, item_body, sq_acc)

        pltpu.sync_copy(qbuf_v, q_hbm.at[pl.ds(base + c * CHUNK, CHUNK)])

    sq_v[...] = sq_acc
    pltpu.sync_copy(sq_v, sq_hbm.at[wid])


_sc_call = functools.partial(
    pl.kernel,
    out_type=(
        jax.ShapeDtypeStruct((B2, LANES), jnp.float32),
        jax.ShapeDtypeStruct((NW, LANES), jnp.float32),
    ),
    mesh=plsc.VectorSubcoreMesh(
        core_axis_name="c", subcore_axis_name="s",
        num_cores=NC, num_subcores=NS),
    scratch_types=[
        pltpu.VMEM((PER_W,), jnp.int32),
        pltpu.VMEM((PER_W,), jnp.int32),
        pltpu.VMEM((PER_W,), jnp.int32),
        pltpu.VMEM((CHUNK, HIDDEN), jnp.float32),
        pltpu.VMEM((CHUNK, HIDDEN), jnp.float32),
        pltpu.VMEM((CHUNK, HIDDEN), jnp.float32),
        pltpu.VMEM((CHUNK, LANES), jnp.float32),
        pltpu.VMEM((LANES,), jnp.float32),
        pltpu.SemaphoreType.DMA,
        pltpu.SemaphoreType.DMA,
        pltpu.SemaphoreType.DMA,
    ],
    compiler_params=pltpu.CompilerParams(use_tc_tiling_on_sc=True),
)(_sc_kernel)


def _tc_kernel(q_ref, y_ref, sq_ref, o_ref):
    row = lax.broadcasted_iota(jnp.int32, (128, 8), 0)
    col = lax.broadcasted_iota(jnp.int32, (128, 8), 1)
    fold = (row // LANES == col).astype(jnp.float32)
    scores = jnp.dot(q_ref[...], fold,
                     preferred_element_type=jnp.float32)
    x = -y_ref[...] * scores
    sp = jnp.maximum(x, 0.0) + jnp.log1p(jnp.exp(-jnp.abs(x)))
    loss = jnp.sum(sp) / BATCH
    regul = jnp.sum(sq_ref[...]) / (BATCH * HIDDEN)
    o_ref[0, 0] = loss + LMBDA * regul


_tc_call = pl.pallas_call(
    _tc_kernel,
    out_shape=jax.ShapeDtypeStruct((1, 1), jnp.float32),
    out_specs=pl.BlockSpec(memory_space=pltpu.SMEM),
)


def kernel(pos_h, pos_t, pos_r, neg_h, neg_t, neg_r, pos_y, neg_y,
           ent_emb, rel_emb):
    h_all = jnp.concatenate([pos_h, neg_h]).astype(jnp.int32)
    t_all = jnp.concatenate([pos_t, neg_t]).astype(jnp.int32)
    r_all = jnp.concatenate([pos_r, neg_r]).astype(jnp.int32)
    y_all = jnp.concatenate([pos_y, neg_y]).reshape(B2 // 8, 8)

    q, sq = _sc_call(h_all, t_all, r_all, ent_emb, rel_emb)
    loss = _tc_call(q.reshape(B2 // 8, 128), y_all,
                    sq.reshape(NW * LANES // 128, 128))
    return loss[0, 0]


# speedup vs baseline: 3.9106x; 1.0361x over previous
import functools

import jax
import jax.numpy as jnp
from jax import lax
from jax.experimental import pallas as pl
from jax.experimental.pallas import tpu as pltpu
from jax.experimental.pallas import tpu_sc as plsc

HIDDEN = 64
BATCH = 16384
LMBDA = 0.0001
NC = 2
NS = 16
NW = NC * NS
LANES = 16
B2 = 2 * BATCH
PER_W = B2 // NW
CHUNK = 128
NCHUNK = PER_W // CHUNK
KH = HIDDEN // LANES


def _sc_kernel(h_hbm, t_hbm, r_hbm, ent_hbm, rel_hbm,
               q_hbm, sq_hbm,
               hidx_v, tidx_v, ridx_v,
               hrow0_v, trow0_v, rrow0_v,
               hrow1_v, trow1_v, rrow1_v,
               qbuf_v, sq_v,
               sa0, sa1, sa2, sb0, sb1, sb2):
    wid = lax.axis_index("s") * NC + lax.axis_index("c")
    base = wid * PER_W

    pltpu.sync_copy(h_hbm.at[pl.ds(base, PER_W)], hidx_v)
    pltpu.sync_copy(t_hbm.at[pl.ds(base, PER_W)], tidx_v)
    pltpu.sync_copy(r_hbm.at[pl.ds(base, PER_W)], ridx_v)

    sets = ((hrow0_v, trow0_v, rrow0_v, sa0, sa1, sa2),
            (hrow1_v, trow1_v, rrow1_v, sb0, sb1, sb2))

    def fire(c, st):
        hrow_v, trow_v, rrow_v, sem0, sem1, sem2 = st

        def fire_body(jj, _):
            hv = hidx_v[pl.ds(c * CHUNK + jj * LANES, LANES)]
            tv = tidx_v[pl.ds(c * CHUNK + jj * LANES, LANES)]
            rv = ridx_v[pl.ds(c * CHUNK + jj * LANES, LANES)]
            for u in range(LANES):
                j = jj * LANES + u
                pltpu.async_copy(ent_hbm.at[hv[u]], hrow_v.at[j], sem0)
                pltpu.async_copy(ent_hbm.at[tv[u]], trow_v.at[j], sem1)
                pltpu.async_copy(rel_hbm.at[rv[u]], rrow_v.at[j], sem2)
            return 0

        lax.fori_loop(0, CHUNK // LANES, fire_body, 0)

    def drain(st):
        hrow_v, trow_v, rrow_v, sem0, sem1, sem2 = st
        dummy = ent_hbm.at[pl.ds(0, CHUNK)]
        pltpu.make_async_copy(dummy, hrow_v, sem0).wait()
        pltpu.make_async_copy(dummy, trow_v, sem1).wait()
        pltpu.make_async_copy(dummy, rrow_v, sem2).wait()

    sq_acc = jnp.zeros((LANES,), jnp.float32)
    fire(0, sets[0])

    for c in range(NCHUNK):
        st = sets[c % 2]
        hrow_v, trow_v, rrow_v = st[0], st[1], st[2]
        drain(st)
        if c + 1 < NCHUNK:
            fire(c + 1, sets[(c + 1) % 2])

        def item_body(i, sq):
            q = jnp.zeros((LANES,), jnp.float32)
            for k in range(KH):
                h = hrow_v[i, pl.ds(k * LANES, LANES)]
                t = trow_v[i, pl.ds(k * LANES, LANES)]
                r = rrow_v[i, pl.ds(k * LANES, LANES)]
                q = q + h * r * t
                sq = sq + h * h + t * t + r * r
            qbuf_v[i, :] = q
            return sq

        sq_acc = lax.fori_loop(0, CHUNK, item_body, sq_acc)

        pltpu.sync_copy(qbuf_v, q_hbm.at[pl.ds(base + c * CHUNK, CHUNK)])

    sq_v[...] = sq_acc
    pltpu.sync_copy(sq_v, sq_hbm.at[wid])


_sc_call = functools.partial(
    pl.kernel,
    out_type=(
        jax.ShapeDtypeStruct((B2, LANES), jnp.float32),
        jax.ShapeDtypeStruct((NW, LANES), jnp.float32),
    ),
    mesh=plsc.VectorSubcoreMesh(
        core_axis_name="c", subcore_axis_name="s",
        num_cores=NC, num_subcores=NS),
    scratch_types=[
        pltpu.VMEM((PER_W,), jnp.int32),
        pltpu.VMEM((PER_W,), jnp.int32),
        pltpu.VMEM((PER_W,), jnp.int32),
        pltpu.VMEM((CHUNK, HIDDEN), jnp.float32),
        pltpu.VMEM((CHUNK, HIDDEN), jnp.float32),
        pltpu.VMEM((CHUNK, HIDDEN), jnp.float32),
        pltpu.VMEM((CHUNK, HIDDEN), jnp.float32),
        pltpu.VMEM((CHUNK, HIDDEN), jnp.float32),
        pltpu.VMEM((CHUNK, HIDDEN), jnp.float32),
        pltpu.VMEM((CHUNK, LANES), jnp.float32),
        pltpu.VMEM((LANES,), jnp.float32),
        pltpu.SemaphoreType.DMA,
        pltpu.SemaphoreType.DMA,
        pltpu.SemaphoreType.DMA,
        pltpu.SemaphoreType.DMA,
        pltpu.SemaphoreType.DMA,
        pltpu.SemaphoreType.DMA,
    ],
    compiler_params=pltpu.CompilerParams(use_tc_tiling_on_sc=True),
)(_sc_kernel)


def _tc_kernel(q_ref, y_ref, sq_ref, o_ref):
    row = lax.broadcasted_iota(jnp.int32, (128, 8), 0)
    col = lax.broadcasted_iota(jnp.int32, (128, 8), 1)
    fold = (row // LANES == col).astype(jnp.float32)
    scores = jnp.dot(q_ref[...], fold,
                     preferred_element_type=jnp.float32)
    x = -y_ref[...] * scores
    sp = jnp.maximum(x, 0.0) + jnp.log1p(jnp.exp(-jnp.abs(x)))
    loss = jnp.sum(sp) / BATCH
    regul = jnp.sum(sq_ref[...]) / (BATCH * HIDDEN)
    o_ref[0, 0] = loss + LMBDA * regul


_tc_call = pl.pallas_call(
    _tc_kernel,
    out_shape=jax.ShapeDtypeStruct((1, 1), jnp.float32),
    out_specs=pl.BlockSpec(memory_space=pltpu.SMEM),
)


def kernel(pos_h, pos_t, pos_r, neg_h, neg_t, neg_r, pos_y, neg_y,
           ent_emb, rel_emb):
    h_all = jnp.concatenate([pos_h, neg_h]).astype(jnp.int32)
    t_all = jnp.concatenate([pos_t, neg_t]).astype(jnp.int32)
    r_all = jnp.concatenate([pos_r, neg_r]).astype(jnp.int32)
    y_all = jnp.concatenate([pos_y, neg_y]).reshape(B2 // 8, 8)

    q, sq = _sc_call(h_all, t_all, r_all, ent_emb, rel_emb)
    loss = _tc_call(q.reshape(B2 // 8, 128), y_all,
                    sq.reshape(NW * LANES // 128, 128))
    return loss[0, 0]
